# baseline (speedup 1.0000x reference)
import functools

import jax
import jax.numpy as jnp
from jax import lax
from jax.experimental import pallas as pl
from jax.experimental.pallas import tpu as pltpu
from jax.experimental.pallas import tpu_sc as plsc

_NC = 2
_NS = 16
_NW = _NC * _NS
_L = 16


def _pe_add_tc_kernel(x_ref, e_ref, o_ref):
    o_ref[...] = x_ref[...] + e_ref[...][None, :, :]


def _tc_kernel(x, embeds):
    B, T, D = x.shape
    bt = 512
    grid = (T // bt,)
    return pl.pallas_call(
        _pe_add_tc_kernel,
        grid=grid,
        in_specs=[
            pl.BlockSpec((B, bt, D), lambda t: (0, t, 0)),
            pl.BlockSpec((bt, D), lambda t: (t, 0)),
        ],
        out_specs=pl.BlockSpec((B, bt, D), lambda t: (0, t, 0)),
        out_shape=jax.ShapeDtypeStruct((B, T, D), x.dtype),
    )(x, embeds)


def _sc_kernel(x, embeds):
    B, T, D = x.shape
    span = T // _NW
    R = 8
    n_chunks = span // R
    mesh = plsc.VectorSubcoreMesh(core_axis_name="c", subcore_axis_name="s")

    @functools.partial(
        pl.kernel,
        out_type=jax.ShapeDtypeStruct((B, T, D), x.dtype),
        mesh=mesh,
        scratch_types=[
            pltpu.VMEM((R, D), jnp.float32),
            pltpu.VMEM((R, D), jnp.float32),
        ],
    )
    def sc_k(x_hbm, e_hbm, out_hbm, xbuf, ebuf):
        wid = lax.axis_index("s") * _NC + lax.axis_index("c")
        t0 = wid * span

        def chunk_body(ci, _):
            tc0 = t0 + ci * R
            pltpu.sync_copy(e_hbm.at[pl.ds(tc0, R)], ebuf)

            def b_body(b, _):
                pltpu.sync_copy(x_hbm.at[b, pl.ds(tc0, R)], xbuf)

                def row_body(r, _):
                    def col_body(c, _):
                        sl = pl.ds(c * _L, _L)
                        xbuf[r, sl] = xbuf[r, sl] + ebuf[r, sl]
                        return 0

                    return lax.fori_loop(0, D // _L, col_body, 0)

                lax.fori_loop(0, R, row_body, 0)
                pltpu.sync_copy(xbuf, out_hbm.at[b, pl.ds(tc0, R)])
                return 0

            lax.fori_loop(0, B, b_body, 0)
            return 0

        lax.fori_loop(0, n_chunks, chunk_body, 0)

    return sc_k(x, embeds)


def kernel(x, embeds):
    return _sc_kernel(x, embeds)


# speedup vs baseline: 5.7419x; 5.7419x over previous
import functools

import jax
import jax.numpy as jnp
from jax import lax
from jax.experimental import pallas as pl
from jax.experimental.pallas import tpu as pltpu
from jax.experimental.pallas import tpu_sc as plsc

_NC = 2
_NS = 16
_NW = _NC * _NS
_L = 16


def _pe_add_tc_kernel(x_ref, e_ref, o_ref):
    o_ref[...] = x_ref[...] + e_ref[...][None, :, :]


def _tc_kernel(x, embeds):
    B, T, D = x.shape
    bt = 512
    grid = (T // bt,)
    return pl.pallas_call(
        _pe_add_tc_kernel,
        grid=grid,
        in_specs=[
            pl.BlockSpec((B, bt, D), lambda t: (0, t, 0)),
            pl.BlockSpec((bt, D), lambda t: (t, 0)),
        ],
        out_specs=pl.BlockSpec((B, bt, D), lambda t: (0, t, 0)),
        out_shape=jax.ShapeDtypeStruct((B, T, D), x.dtype),
    )(x, embeds)


def _sc_kernel(x, embeds):
    B, T, D = x.shape
    span = T // _NW
    R = 8
    n_chunks = span // R
    mesh = plsc.VectorSubcoreMesh(core_axis_name="c", subcore_axis_name="s")

    @functools.partial(
        pl.kernel,
        out_type=jax.ShapeDtypeStruct((B, T, D), x.dtype),
        mesh=mesh,
        scratch_types=[
            pltpu.VMEM((R, D), jnp.float32),
            pltpu.VMEM((R, D), jnp.float32),
        ],
    )
    def sc_k(x_hbm, e_hbm, out_hbm, xbuf, ebuf):
        wid = lax.axis_index("s") * _NC + lax.axis_index("c")
        t0 = wid * span

        def chunk_body(ci, _):
            tc0 = t0 + ci * R
            pltpu.sync_copy(e_hbm.at[pl.ds(tc0, R)], ebuf)

            def b_body(b, _):
                pltpu.sync_copy(x_hbm.at[b, pl.ds(tc0, R)], xbuf)

                def row_body(r, _):
                    def col_body(c, _):
                        sl = pl.ds(c * _L, _L)
                        xbuf[r, sl] = xbuf[r, sl] + ebuf[r, sl]
                        return 0

                    return lax.fori_loop(0, D // _L, col_body, 0)

                lax.fori_loop(0, R, row_body, 0)
                pltpu.sync_copy(xbuf, out_hbm.at[b, pl.ds(tc0, R)])
                return 0

            lax.fori_loop(0, B, b_body, 0)
            return 0

        lax.fori_loop(0, n_chunks, chunk_body, 0)

    return sc_k(x, embeds)


def kernel(x, embeds):
    return _tc_kernel(x, embeds)


# speedup vs baseline: 5.7846x; 1.0074x over previous
import functools

import jax
import jax.numpy as jnp
from jax import lax
from jax.experimental import pallas as pl
from jax.experimental.pallas import tpu as pltpu
from jax.experimental.pallas import tpu_sc as plsc

_NC = 2
_NS = 16
_NW = _NC * _NS
_L = 16


def _pe_add_tc_kernel(x_ref, e_ref, o_ref):
    o_ref[...] = x_ref[...] + e_ref[...][None, :, :]


def _tc_kernel(x, embeds):
    B, T, D = x.shape
    bt = 2048
    grid = (T // bt, B)
    return pl.pallas_call(
        _pe_add_tc_kernel,
        grid=grid,
        in_specs=[
            pl.BlockSpec((1, bt, D), lambda t, b: (b, t, 0)),
            pl.BlockSpec((bt, D), lambda t, b: (t, 0)),
        ],
        out_specs=pl.BlockSpec((1, bt, D), lambda t, b: (b, t, 0)),
        out_shape=jax.ShapeDtypeStruct((B, T, D), x.dtype),
    )(x, embeds)


def _sc_kernel(x, embeds):
    B, T, D = x.shape
    span = T // _NW
    R = 8
    n_chunks = span // R
    mesh = plsc.VectorSubcoreMesh(core_axis_name="c", subcore_axis_name="s")

    @functools.partial(
        pl.kernel,
        out_type=jax.ShapeDtypeStruct((B, T, D), x.dtype),
        mesh=mesh,
        scratch_types=[
            pltpu.VMEM((R, D), jnp.float32),
            pltpu.VMEM((R, D), jnp.float32),
        ],
    )
    def sc_k(x_hbm, e_hbm, out_hbm, xbuf, ebuf):
        wid = lax.axis_index("s") * _NC + lax.axis_index("c")
        t0 = wid * span

        def chunk_body(ci, _):
            tc0 = t0 + ci * R
            pltpu.sync_copy(e_hbm.at[pl.ds(tc0, R)], ebuf)

            def b_body(b, _):
                pltpu.sync_copy(x_hbm.at[b, pl.ds(tc0, R)], xbuf)

                def row_body(r, _):
                    def col_body(c, _):
                        sl = pl.ds(c * _L, _L)
                        xbuf[r, sl] = xbuf[r, sl] + ebuf[r, sl]
                        return 0

                    return lax.fori_loop(0, D // _L, col_body, 0)

                lax.fori_loop(0, R, row_body, 0)
                pltpu.sync_copy(xbuf, out_hbm.at[b, pl.ds(tc0, R)])
                return 0

            lax.fori_loop(0, B, b_body, 0)
            return 0

        lax.fori_loop(0, n_chunks, chunk_body, 0)

    return sc_k(x, embeds)


def kernel(x, embeds):
    return _tc_kernel(x, embeds)
